# baseline (speedup 1.0000x reference)
import functools

import jax
import jax.numpy as jnp
from jax import lax
from jax.experimental import pallas as pl
from jax.experimental.pallas import tpu as pltpu
from jax.experimental.pallas import tpu_sc as plsc

_N = 100000
_NP = 100096
_H = 16
_E = 1600000
_NT = 16
_SUB = 128
_CHUNK = 256
_NSUB = _CHUNK // _SUB
_NCH = 400
_EP = _NT * _NCH * _CHUNK
_EROWS = _EP // _SUB
_RPT = _NP // _NT
_RCH = 184
_NR = _RPT // _RCH
_L = 3
_NBUF = 5


def _body(ego0, cols2, rows2, vals, zeros_h, out, egowork,
          accum, colsb, rowsb, valsb, gath, tmp, tmp2,
          sem_i, sem_g, sem_s):
    c = lax.axis_index("c")
    s = lax.axis_index("s")
    nodebase = c * _NP
    rowbase = s * _RPT
    edgerow0 = s * (_NCH * _NSUB)
    splats = [jnp.full((_H,), j, jnp.int32) for j in range(_H)]

    def idx_fire(i, t):
        crow = edgerow0 + i * _NSUB
        pltpu.async_copy(cols2.at[pl.ds(c * _EROWS + crow, _NSUB), :],
                         colsb.at[pl.ds(t * _NSUB, _NSUB), :], sem_i)
        pltpu.async_copy(rows2.at[pl.ds(crow, _NSUB), :],
                         rowsb.at[pl.ds(t * _NSUB, _NSUB), :], sem_i)
        pltpu.async_copy(vals.at[pl.ds(crow * _SUB, _CHUNK)],
                         valsb.at[pl.ds(t * _CHUNK, _CHUNK)], sem_i)

    def idx_wait(t):
        pltpu.make_async_copy(cols2.at[pl.ds(0, _NSUB), :],
                              colsb.at[pl.ds(t * _NSUB, _NSUB), :],
                              sem_i).wait()
        pltpu.make_async_copy(rows2.at[pl.ds(0, _NSUB), :],
                              rowsb.at[pl.ds(t * _NSUB, _NSUB), :],
                              sem_i).wait()
        pltpu.make_async_copy(vals.at[pl.ds(0, _CHUNK)],
                              valsb.at[pl.ds(t * _CHUNK, _CHUNK)],
                              sem_i).wait()

    def gather_fire(t):
        for j in range(_NSUB):
            tj = t * _NSUB + j
            pltpu.async_copy(egowork.at[colsb.at[tj]],
                             gath.at[pl.ds(tj * _SUB, _SUB), :], sem_g)

    def gather_wait(t):
        for j in range(_NSUB):
            tj = t * _NSUB + j
            pltpu.make_async_copy(egowork.at[colsb.at[tj]],
                                  gath.at[pl.ds(tj * _SUB, _SUB), :],
                                  sem_g).wait()

    def scale_sub(t, j):
        base0 = t * _CHUNK + j * _SUB

        def _grp(g, cc):
            base = base0 + g * _H
            v16 = valsb[pl.ds(base, _H)]
            for k in range(_H):
                bc = v16.at[splats[k]].get(mode="promise_in_bounds")
                gath[base + k, :] = gath[base + k, :] * bc
            return cc

        lax.fori_loop(0, _SUB // _H, _grp, 0)

    def scale_scatter(t):
        for j in range(_NSUB):
            tj = t * _NSUB + j
            scale_sub(t, j)
            pltpu.async_copy(gath.at[pl.ds(tj * _SUB, _SUB), :],
                             accum.at[rowsb.at[tj]], sem_s, add=True)

    def scatter_wait(t):
        for j in range(_NSUB):
            tj = t * _NSUB + j
            pltpu.make_async_copy(gath.at[pl.ds(tj * _SUB, _SUB), :],
                                  accum.at[rowsb.at[tj]], sem_s).wait()

    for r in range(_NR):
        lr = rowbase + r * _RCH
        g0 = nodebase + lr
        pltpu.sync_copy(ego0.at[pl.ds(g0, _RCH), :], tmp)
        pltpu.sync_copy(tmp, egowork.at[pl.ds(g0, _RCH), :])
        pltpu.sync_copy(tmp, out.at[pl.ds(g0, _RCH), :])
        pltpu.sync_copy(zeros_h, accum.at[pl.ds(lr, _RCH), :])
    plsc.subcore_barrier()

    for l in range(_L):
        idx_fire(0, 0)
        idx_fire(1, 1)
        idx_fire(2, 2)
        idx_wait(0)
        gather_fire(0)
        idx_wait(1)
        gather_fire(1)
        idx_fire(3, 3)
        idx_wait(2)
        gather_fire(2)
        gather_wait(0)
        scale_scatter(0)
        idx_fire(4, 4)
        idx_wait(3)
        gather_fire(3)
        gather_wait(1)
        scale_scatter(1)

        def _steady(i, carry):
            t = lax.rem(i, _NBUF)
            t2 = lax.rem(i + 2, _NBUF)
            t3 = lax.rem(i + 3, _NBUF)
            scatter_wait(t3)
            idx_fire(i + 3, t3)
            idx_wait(t2)
            gather_fire(t2)
            gather_wait(t)
            scale_scatter(t)
            return carry

        lax.fori_loop(2, _NCH - 3, _steady, 0)

        scatter_wait(_NCH % _NBUF)
        idx_wait((_NCH - 1) % _NBUF)
        gather_fire((_NCH - 1) % _NBUF)
        gather_wait((_NCH - 3) % _NBUF)
        scale_scatter((_NCH - 3) % _NBUF)
        scatter_wait((_NCH + 1) % _NBUF)
        gather_wait((_NCH - 2) % _NBUF)
        scale_scatter((_NCH - 2) % _NBUF)
        scatter_wait((_NCH + 2) % _NBUF)
        gather_wait((_NCH - 1) % _NBUF)
        scale_scatter((_NCH - 1) % _NBUF)
        scatter_wait((_NCH - 2) % _NBUF)
        scatter_wait((_NCH - 1) % _NBUF)
        plsc.subcore_barrier()

        last = l == _L - 1
        for r in range(_NR):
            lr = rowbase + r * _RCH
            g0 = nodebase + lr
            pltpu.sync_copy(accum.at[pl.ds(lr, _RCH), :], tmp)
            if not last:
                pltpu.sync_copy(zeros_h, accum.at[pl.ds(lr, _RCH), :])
                pltpu.sync_copy(tmp, egowork.at[pl.ds(g0, _RCH), :])
            pltpu.sync_copy(out.at[pl.ds(g0, _RCH), :], tmp2)

            if last:
                def _acc(i, cc):
                    tmp2[i, :] = (tmp2[i, :] + tmp[i, :]) * 0.25
                    return cc
            else:
                def _acc(i, cc):
                    tmp2[i, :] = tmp2[i, :] + tmp[i, :]
                    return cc

            lax.fori_loop(0, _RCH, _acc, 0)
            pltpu.sync_copy(tmp2, out.at[pl.ds(g0, _RCH), :])
        if not last:
            plsc.subcore_barrier()


@functools.cache
def _get_launch():
    return pl.kernel(
        _body,
        out_type=(
            jax.ShapeDtypeStruct((2 * _NP, _H), jnp.float32),
            jax.ShapeDtypeStruct((2 * _NP, _H), jnp.float32),
        ),
        mesh=plsc.VectorSubcoreMesh(core_axis_name="c", subcore_axis_name="s"),
        scratch_types=(
            pltpu.VMEM_SHARED((_NP, _H), jnp.float32),
            pltpu.VMEM((_NBUF * _NSUB, _SUB), jnp.int32),
            pltpu.VMEM((_NBUF * _NSUB, _SUB), jnp.int32),
            pltpu.VMEM((_NBUF * _CHUNK,), jnp.float32),
            pltpu.VMEM((_NBUF * _CHUNK, _H), jnp.float32),
            pltpu.VMEM((_RCH, _H), jnp.float32),
            pltpu.VMEM((_RCH, _H), jnp.float32),
            pltpu.SemaphoreType.DMA,
            pltpu.SemaphoreType.DMA,
            pltpu.SemaphoreType.DMA,
        ),
        compiler_params=pltpu.CompilerParams(use_tc_tiling_on_sc=False),
    )


def kernel(user_emb, item_emb, adj_vals, adj_rows, adj_cols):
    n_user = user_emb.shape[0]
    ego = jnp.concatenate([user_emb, item_emb], axis=0)
    npad = jnp.zeros((_NP - _N, _H), jnp.float32)
    ego0 = jnp.concatenate([ego[:, :_H], npad, ego[:, _H:], npad], axis=0)
    pad = _EP - _E
    vals_p = jnp.concatenate([adj_vals, jnp.zeros((pad,), jnp.float32)])
    rows_p = jnp.concatenate([adj_rows, jnp.zeros((pad,), jnp.int32)])
    cols_p = jnp.concatenate([adj_cols, jnp.zeros((pad,), jnp.int32)])
    cols2 = jnp.concatenate([cols_p, cols_p + _NP]).reshape(2 * _EROWS, _SUB)
    rows2 = rows_p.reshape(_EROWS, _SUB)

    zeros_h = jnp.zeros((_RCH, _H), jnp.float32)
    mean_halves, _ = _get_launch()(ego0, cols2, rows2, vals_p, zeros_h)
    mean = jnp.concatenate(
        [mean_halves[:_N], mean_halves[_NP:_NP + _N]], axis=1)
    return mean[:n_user], mean[n_user:]


# speedup vs baseline: 1.1566x; 1.1566x over previous
import functools

import jax
import jax.numpy as jnp
from jax import lax
from jax.experimental import pallas as pl
from jax.experimental.pallas import tpu as pltpu
from jax.experimental.pallas import tpu_sc as plsc

_N = 100000
_NP = 100096
_H = 16
_E = 1600000
_NT = 16
_SUB = 128
_CHUNK = 256
_NSUB = _CHUNK // _SUB
_NCH = 400
_EP = _NT * _NCH * _CHUNK
_EROWS = _EP // _SUB
_RPT = _NP // _NT
_RCH = 272
_NR = _RPT // _RCH
_L = 3
_NBUF = 4


def _body(ego0, cols2, rows2, vals, zeros_h, out, egowork,
          accum, colsb, rowsb, valsb, gath, tmp, tmp2,
          sem_i, sem_g, sem_s):
    c = lax.axis_index("c")
    s = lax.axis_index("s")
    nodebase = c * _NP
    rowbase = s * _RPT
    edgerow0 = s * (_NCH * _NSUB)
    splats = [jnp.full((_H,), j, jnp.int32) for j in range(_H)]

    def idx_fire(i, t):
        crow = edgerow0 + i * _NSUB
        pltpu.async_copy(cols2.at[pl.ds(c * _EROWS + crow, _NSUB), :],
                         colsb.at[pl.ds(t * _NSUB, _NSUB), :], sem_i)
        pltpu.async_copy(rows2.at[pl.ds(crow, _NSUB), :],
                         rowsb.at[pl.ds(t * _NSUB, _NSUB), :], sem_i)
        pltpu.async_copy(vals.at[pl.ds(crow * _SUB, _CHUNK)],
                         valsb.at[pl.ds(t * _CHUNK, _CHUNK)], sem_i)

    def idx_wait(t):
        pltpu.make_async_copy(cols2.at[pl.ds(0, _NSUB), :],
                              colsb.at[pl.ds(t * _NSUB, _NSUB), :],
                              sem_i).wait()
        pltpu.make_async_copy(rows2.at[pl.ds(0, _NSUB), :],
                              rowsb.at[pl.ds(t * _NSUB, _NSUB), :],
                              sem_i).wait()
        pltpu.make_async_copy(vals.at[pl.ds(0, _CHUNK)],
                              valsb.at[pl.ds(t * _CHUNK, _CHUNK)],
                              sem_i).wait()

    def gather_fire(t):
        for j in range(_NSUB):
            tj = t * _NSUB + j
            pltpu.async_copy(egowork.at[colsb.at[tj]],
                             gath.at[pl.ds(tj * _SUB, _SUB), :], sem_g)

    def gather_wait(t):
        for j in range(_NSUB):
            tj = t * _NSUB + j
            pltpu.make_async_copy(egowork.at[colsb.at[tj]],
                                  gath.at[pl.ds(tj * _SUB, _SUB), :],
                                  sem_g).wait()

    def scale_sub(t, j):
        base0 = t * _CHUNK + j * _SUB

        def _grp(g, cc):
            base = base0 + g * _H
            v16 = valsb[pl.ds(base, _H)]
            for k in range(_H):
                bc = v16.at[splats[k]].get(mode="promise_in_bounds")
                gath[base + k, :] = gath[base + k, :] * bc
            return cc

        lax.fori_loop(0, _SUB // _H, _grp, 0)

    def scale_scatter(t):
        for j in range(_NSUB):
            tj = t * _NSUB + j
            scale_sub(t, j)
            pltpu.async_copy(gath.at[pl.ds(tj * _SUB, _SUB), :],
                             accum.at[rowsb.at[tj]], sem_s, add=True)

    def scatter_wait(t):
        for j in range(_NSUB):
            tj = t * _NSUB + j
            pltpu.make_async_copy(gath.at[pl.ds(tj * _SUB, _SUB), :],
                                  accum.at[rowsb.at[tj]], sem_s).wait()

    for r in range(_NR):
        lr = rowbase + r * _RCH
        g0 = nodebase + lr
        pltpu.sync_copy(ego0.at[pl.ds(g0, _RCH), :], tmp)
        pltpu.sync_copy(tmp, egowork.at[pl.ds(g0, _RCH), :])
        pltpu.sync_copy(tmp, out.at[pl.ds(g0, _RCH), :])
        pltpu.sync_copy(zeros_h, accum.at[pl.ds(lr, _RCH), :])
    plsc.subcore_barrier()

    for l in range(_L):
        idx_fire(0, 0)
        idx_fire(1, 1)
        idx_wait(0)
        gather_fire(0)
        idx_wait(1)
        gather_fire(1)
        gather_wait(0)
        idx_fire(2, 2)
        scale_scatter(0)
        idx_wait(2)
        gather_fire(2)
        gather_wait(1)
        idx_fire(3, 3)
        scale_scatter(1)

        def _steady(i, carry):
            t = lax.rem(i, _NBUF)
            tn = lax.rem(i + 1, _NBUF)
            tp = lax.rem(i + 2, _NBUF)
            idx_wait(tn)
            gather_fire(tn)
            gather_wait(t)
            scatter_wait(tp)
            idx_fire(i + 2, tp)
            scale_scatter(t)
            return carry

        lax.fori_loop(2, _NCH - 2, _steady, 0)

        idx_wait((_NCH - 1) % _NBUF)
        gather_fire((_NCH - 1) % _NBUF)
        gather_wait((_NCH - 2) % _NBUF)
        scatter_wait(_NCH % _NBUF)
        scale_scatter((_NCH - 2) % _NBUF)
        gather_wait((_NCH - 1) % _NBUF)
        scatter_wait((_NCH + 1) % _NBUF)
        scale_scatter((_NCH - 1) % _NBUF)
        scatter_wait((_NCH - 2) % _NBUF)
        scatter_wait((_NCH - 1) % _NBUF)
        plsc.subcore_barrier()

        last = l == _L - 1
        for r in range(_NR):
            lr = rowbase + r * _RCH
            g0 = nodebase + lr
            pltpu.sync_copy(accum.at[pl.ds(lr, _RCH), :], tmp)
            if not last:
                pltpu.sync_copy(zeros_h, accum.at[pl.ds(lr, _RCH), :])
                pltpu.sync_copy(tmp, egowork.at[pl.ds(g0, _RCH), :])
            pltpu.sync_copy(out.at[pl.ds(g0, _RCH), :], tmp2)

            if last:
                def _acc(i, cc):
                    tmp2[i, :] = (tmp2[i, :] + tmp[i, :]) * 0.25
                    return cc
            else:
                def _acc(i, cc):
                    tmp2[i, :] = tmp2[i, :] + tmp[i, :]
                    return cc

            lax.fori_loop(0, _RCH, _acc, 0)
            pltpu.sync_copy(tmp2, out.at[pl.ds(g0, _RCH), :])
        if not last:
            plsc.subcore_barrier()


@functools.cache
def _get_launch():
    return pl.kernel(
        _body,
        out_type=(
            jax.ShapeDtypeStruct((2 * _NP, _H), jnp.float32),
            jax.ShapeDtypeStruct((2 * _NP, _H), jnp.float32),
        ),
        mesh=plsc.VectorSubcoreMesh(core_axis_name="c", subcore_axis_name="s"),
        scratch_types=(
            pltpu.VMEM_SHARED((_NP, _H), jnp.float32),
            pltpu.VMEM((_NBUF * _NSUB, _SUB), jnp.int32),
            pltpu.VMEM((_NBUF * _NSUB, _SUB), jnp.int32),
            pltpu.VMEM((_NBUF * _CHUNK,), jnp.float32),
            pltpu.VMEM((_NBUF * _CHUNK, _H), jnp.float32),
            pltpu.VMEM((_RCH, _H), jnp.float32),
            pltpu.VMEM((_RCH, _H), jnp.float32),
            pltpu.SemaphoreType.DMA,
            pltpu.SemaphoreType.DMA,
            pltpu.SemaphoreType.DMA,
        ),
        compiler_params=pltpu.CompilerParams(use_tc_tiling_on_sc=False),
    )


def kernel(user_emb, item_emb, adj_vals, adj_rows, adj_cols):
    n_user = user_emb.shape[0]
    ego = jnp.concatenate([user_emb, item_emb], axis=0)
    npad = jnp.zeros((_NP - _N, _H), jnp.float32)
    ego0 = jnp.concatenate([ego[:, :_H], npad, ego[:, _H:], npad], axis=0)
    pad = _EP - _E
    vals_p = jnp.concatenate([adj_vals, jnp.zeros((pad,), jnp.float32)])
    rows_p = jnp.concatenate([adj_rows, jnp.zeros((pad,), jnp.int32)])
    cols_p = jnp.concatenate([adj_cols, jnp.zeros((pad,), jnp.int32)])
    cols2 = jnp.concatenate([cols_p, cols_p + _NP]).reshape(2 * _EROWS, _SUB)
    rows2 = rows_p.reshape(_EROWS, _SUB)

    zeros_h = jnp.zeros((_RCH, _H), jnp.float32)
    mean_halves, _ = _get_launch()(ego0, cols2, rows2, vals_p, zeros_h)
    mean = jnp.concatenate(
        [mean_halves[:_N], mean_halves[_NP:_NP + _N]], axis=1)
    return mean[:n_user], mean[n_user:]
